# baseline (speedup 1.0000x reference)
import functools

import jax
import jax.numpy as jnp
from jax import lax
from jax.experimental import pallas as pl
from jax.experimental.pallas import tpu as pltpu
from jax.experimental.pallas import tpu_sc as plsc

D_MODEL = 1024
LANES = 16


@functools.lru_cache(maxsize=None)
def _build_sc_kernel(n_rows: int, d: int, vocab: int):
    info = plsc.get_sparse_core_info()
    nc, ns = info.num_cores, info.num_subcores
    nw = nc * ns
    assert n_rows % nw == 0
    rows_per_w = n_rows // nw
    chunk = 32
    steps = rows_per_w // chunk
    vecs_per_row = d // LANES

    mesh = plsc.VectorSubcoreMesh(core_axis_name="c", subcore_axis_name="s")

    @functools.partial(
        pl.kernel,
        mesh=mesh,
        out_type=jax.ShapeDtypeStruct((n_rows, d), jnp.float32),
        scratch_types=[
            pltpu.VMEM((chunk,), jnp.int32),
            pltpu.VMEM((chunk, d), jnp.float32),
            pltpu.VMEM((chunk, d), jnp.float32),
            pltpu.SemaphoreType.DMA,
        ],
    )
    def k(x_hbm, idx_hbm, pe_hbm, out_hbm, idx_v, pe_v, x_v, sem):
        wid = lax.axis_index("s") * nc + lax.axis_index("c")
        base = wid * rows_per_w

        def step(s, carry):
            r0 = base + s * chunk
            pltpu.sync_copy(idx_hbm.at[pl.ds(r0, chunk)], idx_v)
            gather = pltpu.async_copy(pe_hbm.at[idx_v], pe_v, sem)
            pltpu.sync_copy(x_hbm.at[pl.ds(r0, chunk)], x_v)
            gather.wait()

            def add_row(r, carry2):
                for c in range(vecs_per_row):
                    sl = pl.ds(c * LANES, LANES)
                    x_v[r, sl] = x_v[r, sl] + pe_v[r, sl]
                return carry2

            lax.fori_loop(0, chunk, add_row, 0, unroll=False)
            pltpu.sync_copy(x_v, out_hbm.at[pl.ds(r0, chunk)])
            return carry

        lax.fori_loop(0, steps, step, 0, unroll=False)

    return k


def kernel(x, rel_times, pe_table):
    b, t, d = x.shape
    n = b * t
    xf = x.reshape(n, d)
    idx = rel_times.reshape(n).astype(jnp.int32)
    out = _build_sc_kernel(n, d, pe_table.shape[0])(xf, idx, pe_table)
    return out.reshape(b, t, d)
